# baseline (speedup 1.0000x reference)
import functools

import jax
import jax.numpy as jnp
from jax import lax
from jax.experimental import pallas as pl
from jax.experimental.pallas import tpu as pltpu
from jax.experimental.pallas import tpu_sc as plsc

N = 10000
E = 160000
E_LG = 320000
F_IN = 256
H_DIM = 512
OUT = 256

RB_E = 1600
RB_N = 2000


def _leaky(x):
    return jnp.where(x >= 0, x, 0.2 * x)


def _k1_body(et_ref, w_ref, out_ref):
    out_ref[...] = _leaky(jnp.dot(et_ref[...], w_ref[...],
                                  preferred_element_type=jnp.float32))


def k1_h(et, W_tin):
    R = RB_E
    return pl.pallas_call(
        _k1_body,
        grid=(E // R,),
        in_specs=[
            pl.BlockSpec((R, 64), lambda i: (i, 0)),
            pl.BlockSpec((64, H_DIM), lambda i: (0, 0)),
        ],
        out_specs=pl.BlockSpec((R, H_DIM), lambda i: (i, 0)),
        out_shape=jax.ShapeDtypeStruct((E, H_DIM), jnp.float32),
    )(et, W_tin)


def _k2_body(agg_ref, deg_ref, h_ref, wnb_ref, wself_ref, *out_refs):
    rrecip = 1.0 / jnp.maximum(deg_ref[:, 0:1], 1.0)
    aggn = agg_ref[...] * rrecip
    h = h_ref[...]
    for c in range(4):
        cs = slice(c * 128, (c + 1) * 128)
        out_refs[c][...] = jnp.maximum(
            jnp.dot(aggn, wnb_ref[:, cs], preferred_element_type=jnp.float32)
            + jnp.dot(h, wself_ref[:, cs], preferred_element_type=jnp.float32),
            0.0)


def k2_tsae(agg, deg16, h, W_tnb, W_tself, epad):
    R = RB_E
    return pl.pallas_call(
        _k2_body,
        grid=(E // R,),
        in_specs=[
            pl.BlockSpec((R, H_DIM), lambda i: (i, 0)),
            pl.BlockSpec((R, 16), lambda i: (i, 0)),
            pl.BlockSpec((R, H_DIM), lambda i: (i, 0)),
            pl.BlockSpec((H_DIM, H_DIM), lambda i: (0, 0)),
            pl.BlockSpec((H_DIM, H_DIM), lambda i: (0, 0)),
        ],
        out_specs=[pl.BlockSpec((R, 128), lambda i: (i, 0))] * 4,
        out_shape=[jax.ShapeDtypeStruct((epad, 128), jnp.float32)] * 4,
    )(agg, deg16, h, W_tnb, W_tself)


def _k3_body(p_ref, wetn_ref, weg_ref, out_ref):
    acc = None
    for c in range(4):
        nfe_c = p_ref[0, c] + p_ref[1, c]
        ks = slice(c * 128, (c + 1) * 128)
        term = jnp.dot(nfe_c, wetn_ref[ks, :], preferred_element_type=jnp.float32)
        acc = term if acc is None else acc + term
    t = _leaky(acc)
    out_ref[...] = jnp.dot(t, weg_ref[...], preferred_element_type=jnp.float32)


def k3_edge_repr(P, W_etn, W_eg):
    R = RB_N
    return pl.pallas_call(
        _k3_body,
        grid=(N // R,),
        in_specs=[
            pl.BlockSpec((2, 4, R, 128), lambda i: (0, 0, i, 0)),
            pl.BlockSpec((H_DIM, H_DIM), lambda i: (0, 0)),
            pl.BlockSpec((H_DIM, H_DIM), lambda i: (0, 0)),
        ],
        out_specs=pl.BlockSpec((R, H_DIM), lambda i: (i, 0)),
        out_shape=jax.ShapeDtypeStruct((N, H_DIM), jnp.float32),
    )(P, W_etn, W_eg)


def _k4_body(x_ref, q_ref, rdeg_ref, wself_ref, wnb_ref, out_ref, *outch_refs):
    rrecip = 1.0 / jnp.maximum(rdeg_ref[0, :, 0:1] + rdeg_ref[1, :, 0:1], 1.0)
    acc = jnp.dot(x_ref[...], wself_ref[...], preferred_element_type=jnp.float32)
    for c in range(2):
        m1_c = (q_ref[0, c] + q_ref[1, c]) * rrecip
        ks = slice(c * 128, (c + 1) * 128)
        acc = acc + jnp.dot(m1_c, wnb_ref[ks, :], preferred_element_type=jnp.float32)
    h1 = jnp.maximum(acc, 0.0)
    out_ref[...] = h1
    for c in range(4):
        outch_refs[c][...] = h1[:, c * 128:(c + 1) * 128]


def k4_h1(x, Q, rdeg, W_s1_self, W_s1_nb):
    R = RB_N
    return pl.pallas_call(
        _k4_body,
        grid=(N // R,),
        in_specs=[
            pl.BlockSpec((R, F_IN), lambda i: (i, 0)),
            pl.BlockSpec((2, 2, R, 128), lambda i: (0, 0, i, 0)),
            pl.BlockSpec((2, R, 128), lambda i: (0, i, 0)),
            pl.BlockSpec((F_IN, H_DIM), lambda i: (0, 0)),
            pl.BlockSpec((F_IN, H_DIM), lambda i: (0, 0)),
        ],
        out_specs=[pl.BlockSpec((R, H_DIM), lambda i: (i, 0))]
        + [pl.BlockSpec((R, 128), lambda i: (i, 0))] * 4,
        out_shape=[jax.ShapeDtypeStruct((N, H_DIM), jnp.float32)]
        + [jax.ShapeDtypeStruct((N, 128), jnp.float32)] * 4,
    )(x, Q, rdeg, W_s1_self, W_s1_nb)


def _k5_body(h1_ref, r_ref, rdeg_ref, er_ref, wself_ref, wnb_ref,
             an_ref, ae_ref, wmix_ref, wout_ref, out_ref):
    rrecip = 1.0 / jnp.maximum(rdeg_ref[0, :, 0:1] + rdeg_ref[1, :, 0:1], 1.0)
    acc = jnp.dot(h1_ref[...], wself_ref[...], preferred_element_type=jnp.float32)
    for c in range(4):
        m2_c = (r_ref[0, c] + r_ref[1, c]) * rrecip
        ks = slice(c * 128, (c + 1) * 128)
        acc = acc + jnp.dot(m2_c, wnb_ref[ks, :], preferred_element_type=jnp.float32)
    node = jnp.maximum(acc, 0.0)
    edge = er_ref[...]
    sn = jnp.dot(node, an_ref[...], preferred_element_type=jnp.float32)
    se = jnp.dot(edge, ae_ref[...], preferred_element_type=jnp.float32)
    m = jnp.maximum(sn, se)
    en = jnp.exp(sn - m)
    ee = jnp.exp(se - m)
    inv = 1.0 / (en + ee)
    mixed = (en * inv) * node + (ee * inv) * edge
    o = jnp.dot(mixed, wmix_ref[...], preferred_element_type=jnp.float32)
    logits = jnp.dot(o, wout_ref[...], preferred_element_type=jnp.float32)
    mx = jnp.max(logits, axis=1, keepdims=True)
    lse = mx + jnp.log(jnp.sum(jnp.exp(logits - mx), axis=1, keepdims=True))
    out_ref[...] = logits - lse


def k5_head(h1, Rm2, rdeg, edge_repr, W_s2_self, W_s2_nb, a_n, a_e, W_mix, W_out):
    a_n = a_n.reshape(H_DIM, 1)
    a_e = a_e.reshape(H_DIM, 1)
    R = RB_N
    return pl.pallas_call(
        _k5_body,
        grid=(N // R,),
        in_specs=[
            pl.BlockSpec((R, H_DIM), lambda i: (i, 0)),
            pl.BlockSpec((2, 4, R, 128), lambda i: (0, 0, i, 0)),
            pl.BlockSpec((2, R, 128), lambda i: (0, i, 0)),
            pl.BlockSpec((R, H_DIM), lambda i: (i, 0)),
            pl.BlockSpec((H_DIM, H_DIM), lambda i: (0, 0)),
            pl.BlockSpec((H_DIM, H_DIM), lambda i: (0, 0)),
            pl.BlockSpec((H_DIM, 1), lambda i: (0, 0)),
            pl.BlockSpec((H_DIM, 1), lambda i: (0, 0)),
            pl.BlockSpec((H_DIM, H_DIM), lambda i: (0, 0)),
            pl.BlockSpec((H_DIM, OUT), lambda i: (0, 0)),
        ],
        out_specs=pl.BlockSpec((R, OUT), lambda i: (i, 0)),
        out_shape=jax.ShapeDtypeStruct((N, OUT), jnp.float32),
    )(h1, Rm2, rdeg, edge_repr, W_s2_self, W_s2_nb, a_n, a_e, W_mix, W_out)


NTILE = 32
EBLK = 128
ZBLK = 64
NR_ACC = 10240
ZPT = NR_ACC // 16


def _pad_idx(idx, n_pad, pad_val):
    idx = idx.astype(jnp.int32)
    pad = jnp.full((n_pad - idx.shape[0],), pad_val, jnp.int32)
    return jnp.concatenate([idx, pad])


def _sc_gather_segsum(nch, want_deg, ept, linear2):
    mesh = plsc.VectorSubcoreMesh(core_axis_name="c", subcore_axis_name="s")
    nblk = ept // EBLK
    npass = nch + (1 if want_deg else 0)

    def body(*refs):
        tbls = refs[:nch]
        (src_hbm, dst_hbm, dst2_hbm, zeros_hbm, ones_hbm, p_hbm, deg_hbm,
         acc, srcv, dstv, dstv2, rows, zbuf, sem) = refs[nch:]
        core = lax.axis_index("c")
        sid = lax.axis_index("s")
        wid = core * 16 + sid
        row0 = pl.multiple_of(sid * ZPT, 8)
        ebase = pl.multiple_of(wid * ept, 8)
        pltpu.sync_copy(zeros_hbm, zbuf)
        for c in range(npass):
            is_deg = want_deg and c == nch
            for r in range(ZPT // ZBLK):
                pltpu.sync_copy(zbuf, acc.at[pl.ds(row0 + r * ZBLK, ZBLK)])
            if is_deg:
                pltpu.sync_copy(ones_hbm, rows)
            plsc.subcore_barrier()

            def blk(j, carry):
                eoff = pl.multiple_of(ebase + j * EBLK, 8)
                pltpu.sync_copy(dst_hbm.at[pl.ds(eoff, EBLK)], dstv)
                if not is_deg:
                    if linear2:
                        pltpu.sync_copy(tbls[c].at[pl.ds(eoff, EBLK)], rows)
                    else:
                        pltpu.sync_copy(src_hbm.at[pl.ds(eoff, EBLK)], srcv)
                        pltpu.async_copy(tbls[c].at[srcv], rows, sem).wait()
                pltpu.sync_copy(rows, acc.at[dstv], add=True)
                if linear2 and not is_deg:
                    pltpu.sync_copy(dst2_hbm.at[pl.ds(eoff, EBLK)], dstv2)
                    pltpu.sync_copy(rows, acc.at[dstv2], add=True)
                return carry

            lax.fori_loop(0, nblk, blk, 0)
            plsc.subcore_barrier()
            if is_deg:
                doff = pl.multiple_of(core * NR_ACC + row0, 8)
                pltpu.sync_copy(acc.at[pl.ds(row0, ZPT)],
                                deg_hbm.at[pl.ds(doff, ZPT)])
            else:
                poff = pl.multiple_of((core * nch + c) * NR_ACC + row0, 8)
                pltpu.sync_copy(acc.at[pl.ds(row0, ZPT)],
                                p_hbm.at[pl.ds(poff, ZPT)])

    out_type = [jax.ShapeDtypeStruct((2 * nch * NR_ACC, 128), jnp.float32),
                jax.ShapeDtypeStruct((2 * NR_ACC, 128), jnp.float32)]
    scratch = [
        pltpu.VMEM_SHARED((NR_ACC, 128), jnp.float32),
        pltpu.VMEM((EBLK,), jnp.int32),
        pltpu.VMEM((EBLK,), jnp.int32),
        pltpu.VMEM((EBLK,), jnp.int32),
        pltpu.VMEM((EBLK, 128), jnp.float32),
        pltpu.VMEM((ZBLK, 128), jnp.float32),
        pltpu.SemaphoreType.DMA,
    ]
    return mesh, body, out_type, scratch


@functools.partial(jax.jit, static_argnums=(4, 5))
def _sc_segsum_call(tbls, src_pad, dst_pad, dst2_pad, want_deg, linear2):
    nch = len(tbls)
    ept = dst_pad.shape[0] // NTILE
    mesh, body, out_type, scratch = _sc_gather_segsum(nch, want_deg, ept,
                                                      linear2)
    zeros = jnp.zeros((ZBLK, 128), jnp.float32)
    ones = jnp.ones((EBLK, 128), jnp.float32)
    k = pl.kernel(body, out_type=out_type, mesh=mesh, scratch_types=scratch)
    p, deg = k(*tbls, src_pad, dst_pad, dst2_pad, zeros, ones)
    return (p.reshape(2, nch, NR_ACC, 128), deg.reshape(2, NR_ACC, 128))


def _segsum_rows(rows, dst, num_segments):
    return jnp.zeros((num_segments, rows.shape[1]), jnp.float32).at[dst].add(rows)


def kernel(x, et, H, raw_edge_index, lg_edge_index, W_tin, W_tself, W_tnb,
           W_etn, W_eg, W_s1_self, W_s1_nb, W_s2_self, W_s2_nb, a_n, a_e,
           W_mix, W_out):
    n_epad = NTILE * EBLK * (-(-E // (NTILE * EBLK)))

    h = k1_h(et, W_tin)

    src = lg_edge_index[0]
    dst = lg_edge_index[1]
    agg = _segsum_rows(h[src], dst, E)
    deg = jnp.zeros((E,), jnp.float32).at[dst].add(1.0)
    deg16 = jnp.broadcast_to(deg[:, None], (E, 16))

    t_ch = k2_tsae(agg, deg16, h, W_tnb, W_tself, n_epad)

    h0_pad = _pad_idx(H[0], n_epad, N)
    h1_pad = _pad_idx(H[1], n_epad, N)
    P, _ = _sc_segsum_call(tuple(t_ch), h0_pad, h0_pad, h1_pad, False, True)

    edge_repr = k3_edge_repr(P, W_etn, W_eg)

    rs = raw_edge_index[0]
    rd = raw_edge_index[1]
    rs_pad = _pad_idx(rs, n_epad, 0)
    rd_pad = _pad_idx(rd, n_epad, N)
    x_ch = x.reshape(N, 2, 128).transpose(1, 0, 2)
    Q, rdeg_p = _sc_segsum_call((x_ch[0], x_ch[1]), rs_pad, rd_pad, rd_pad,
                                True, False)

    h1, *h1_ch = k4_h1(x, Q, rdeg_p, W_s1_self, W_s1_nb)

    Rm2, _ = _sc_segsum_call(tuple(h1_ch), rs_pad, rd_pad, rd_pad,
                             False, False)

    return k5_head(h1, Rm2, rdeg_p, edge_repr, W_s2_self, W_s2_nb,
                   a_n, a_e, W_mix, W_out)


# speedup vs baseline: 1.0926x; 1.0926x over previous
import functools

import jax
import jax.numpy as jnp
from jax import lax
from jax.experimental import pallas as pl
from jax.experimental.pallas import tpu as pltpu
from jax.experimental.pallas import tpu_sc as plsc

N = 10000
E = 160000
E_LG = 320000
F_IN = 256
H_DIM = 512
OUT = 256

RB_E = 1600
RB_N = 2000


def _leaky(x):
    return jnp.where(x >= 0, x, 0.2 * x)


def _k1_body(et_ref, w_ref, out_ref):
    out_ref[...] = _leaky(jnp.dot(et_ref[...], w_ref[...],
                                  preferred_element_type=jnp.float32))


def k1_h(et, W_tin):
    R = RB_E
    return pl.pallas_call(
        _k1_body,
        grid=(E // R,),
        in_specs=[
            pl.BlockSpec((R, 64), lambda i: (i, 0)),
            pl.BlockSpec((64, H_DIM), lambda i: (0, 0)),
        ],
        out_specs=pl.BlockSpec((R, H_DIM), lambda i: (i, 0)),
        out_shape=jax.ShapeDtypeStruct((E, H_DIM), jnp.float32),
    )(et, W_tin)


def _k2_body(agg_ref, deg_ref, h_ref, wnb_ref, wself_ref, *out_refs):
    rrecip = 1.0 / jnp.maximum(deg_ref[:, 0:1], 1.0)
    aggn = agg_ref[...] * rrecip
    h = h_ref[...]
    for c in range(4):
        cs = slice(c * 128, (c + 1) * 128)
        out_refs[c][...] = jnp.maximum(
            jnp.dot(aggn, wnb_ref[:, cs], preferred_element_type=jnp.float32)
            + jnp.dot(h, wself_ref[:, cs], preferred_element_type=jnp.float32),
            0.0)


def k2_tsae(agg, deg16, h, W_tnb, W_tself, epad):
    R = RB_E
    return pl.pallas_call(
        _k2_body,
        grid=(E // R,),
        in_specs=[
            pl.BlockSpec((R, H_DIM), lambda i: (i, 0)),
            pl.BlockSpec((R, 16), lambda i: (i, 0)),
            pl.BlockSpec((R, H_DIM), lambda i: (i, 0)),
            pl.BlockSpec((H_DIM, H_DIM), lambda i: (0, 0)),
            pl.BlockSpec((H_DIM, H_DIM), lambda i: (0, 0)),
        ],
        out_specs=[pl.BlockSpec((R, 128), lambda i: (i, 0))] * 4,
        out_shape=[jax.ShapeDtypeStruct((epad, 128), jnp.float32)] * 4,
    )(agg, deg16, h, W_tnb, W_tself)


def _k3_body(p_ref, wetn_ref, weg_ref, out_ref):
    acc = None
    for c in range(4):
        nfe_c = p_ref[0, c] + p_ref[1, c]
        ks = slice(c * 128, (c + 1) * 128)
        term = jnp.dot(nfe_c, wetn_ref[ks, :], preferred_element_type=jnp.float32)
        acc = term if acc is None else acc + term
    t = _leaky(acc)
    out_ref[...] = jnp.dot(t, weg_ref[...], preferred_element_type=jnp.float32)


def k3_edge_repr(P, W_etn, W_eg):
    R = RB_N
    return pl.pallas_call(
        _k3_body,
        grid=(N // R,),
        in_specs=[
            pl.BlockSpec((2, 4, R, 128), lambda i: (0, 0, i, 0)),
            pl.BlockSpec((H_DIM, H_DIM), lambda i: (0, 0)),
            pl.BlockSpec((H_DIM, H_DIM), lambda i: (0, 0)),
        ],
        out_specs=pl.BlockSpec((R, H_DIM), lambda i: (i, 0)),
        out_shape=jax.ShapeDtypeStruct((N, H_DIM), jnp.float32),
    )(P, W_etn, W_eg)


def _k4_body(x_ref, q_ref, rdeg_ref, wself_ref, wnb_ref, out_ref, *outch_refs):
    rrecip = 1.0 / jnp.maximum(rdeg_ref[0, :, 0:1] + rdeg_ref[1, :, 0:1], 1.0)
    acc = jnp.dot(x_ref[...], wself_ref[...], preferred_element_type=jnp.float32)
    for c in range(2):
        m1_c = (q_ref[0, c] + q_ref[1, c]) * rrecip
        ks = slice(c * 128, (c + 1) * 128)
        acc = acc + jnp.dot(m1_c, wnb_ref[ks, :], preferred_element_type=jnp.float32)
    h1 = jnp.maximum(acc, 0.0)
    out_ref[...] = h1
    for c in range(4):
        outch_refs[c][...] = h1[:, c * 128:(c + 1) * 128]


def k4_h1(x, Q, rdeg, W_s1_self, W_s1_nb):
    R = RB_N
    return pl.pallas_call(
        _k4_body,
        grid=(N // R,),
        in_specs=[
            pl.BlockSpec((R, F_IN), lambda i: (i, 0)),
            pl.BlockSpec((2, 2, R, 128), lambda i: (0, 0, i, 0)),
            pl.BlockSpec((2, R, 128), lambda i: (0, i, 0)),
            pl.BlockSpec((F_IN, H_DIM), lambda i: (0, 0)),
            pl.BlockSpec((F_IN, H_DIM), lambda i: (0, 0)),
        ],
        out_specs=[pl.BlockSpec((R, H_DIM), lambda i: (i, 0))]
        + [pl.BlockSpec((R, 128), lambda i: (i, 0))] * 4,
        out_shape=[jax.ShapeDtypeStruct((N, H_DIM), jnp.float32)]
        + [jax.ShapeDtypeStruct((N, 128), jnp.float32)] * 4,
    )(x, Q, rdeg, W_s1_self, W_s1_nb)


def _k5_body(h1_ref, r_ref, rdeg_ref, er_ref, wself_ref, wnb_ref,
             an_ref, ae_ref, wmix_ref, wout_ref, out_ref):
    rrecip = 1.0 / jnp.maximum(rdeg_ref[0, :, 0:1] + rdeg_ref[1, :, 0:1], 1.0)
    acc = jnp.dot(h1_ref[...], wself_ref[...], preferred_element_type=jnp.float32)
    for c in range(4):
        m2_c = (r_ref[0, c] + r_ref[1, c]) * rrecip
        ks = slice(c * 128, (c + 1) * 128)
        acc = acc + jnp.dot(m2_c, wnb_ref[ks, :], preferred_element_type=jnp.float32)
    node = jnp.maximum(acc, 0.0)
    edge = er_ref[...]
    sn = jnp.dot(node, an_ref[...], preferred_element_type=jnp.float32)
    se = jnp.dot(edge, ae_ref[...], preferred_element_type=jnp.float32)
    m = jnp.maximum(sn, se)
    en = jnp.exp(sn - m)
    ee = jnp.exp(se - m)
    inv = 1.0 / (en + ee)
    mixed = (en * inv) * node + (ee * inv) * edge
    o = jnp.dot(mixed, wmix_ref[...], preferred_element_type=jnp.float32)
    logits = jnp.dot(o, wout_ref[...], preferred_element_type=jnp.float32)
    mx = jnp.max(logits, axis=1, keepdims=True)
    lse = mx + jnp.log(jnp.sum(jnp.exp(logits - mx), axis=1, keepdims=True))
    out_ref[...] = logits - lse


def k5_head(h1, Rm2, rdeg, edge_repr, W_s2_self, W_s2_nb, a_n, a_e, W_mix, W_out):
    a_n = a_n.reshape(H_DIM, 1)
    a_e = a_e.reshape(H_DIM, 1)
    R = RB_N
    return pl.pallas_call(
        _k5_body,
        grid=(N // R,),
        in_specs=[
            pl.BlockSpec((R, H_DIM), lambda i: (i, 0)),
            pl.BlockSpec((2, 4, R, 128), lambda i: (0, 0, i, 0)),
            pl.BlockSpec((2, R, 128), lambda i: (0, i, 0)),
            pl.BlockSpec((R, H_DIM), lambda i: (i, 0)),
            pl.BlockSpec((H_DIM, H_DIM), lambda i: (0, 0)),
            pl.BlockSpec((H_DIM, H_DIM), lambda i: (0, 0)),
            pl.BlockSpec((H_DIM, 1), lambda i: (0, 0)),
            pl.BlockSpec((H_DIM, 1), lambda i: (0, 0)),
            pl.BlockSpec((H_DIM, H_DIM), lambda i: (0, 0)),
            pl.BlockSpec((H_DIM, OUT), lambda i: (0, 0)),
        ],
        out_specs=pl.BlockSpec((R, OUT), lambda i: (i, 0)),
        out_shape=jax.ShapeDtypeStruct((N, OUT), jnp.float32),
    )(h1, Rm2, rdeg, edge_repr, W_s2_self, W_s2_nb, a_n, a_e, W_mix, W_out)


NTILE = 32
EBLK = 64
NR_ACC = 10240
ZPT = NR_ACC // 16


def _pad_idx(idx, n_pad, pad_val):
    idx = idx.astype(jnp.int32)
    pad = jnp.full((n_pad - idx.shape[0],), pad_val, jnp.int32)
    return jnp.concatenate([idx, pad])


def _sc_gather_segsum(nch, want_deg, ept, linear2):
    mesh = plsc.VectorSubcoreMesh(core_axis_name="c", subcore_axis_name="s")
    nblk = ept // EBLK
    assert nblk % 2 == 0
    npass = nch + (1 if want_deg else 0)

    def body(*refs):
        tbls = refs[:nch]
        (src_hbm, dst_hbm, dst2_hbm, zeros_hbm, ones_hbm, p_hbm, deg_hbm,
         acc, srcall, dstall, dst2all, rows0, rows1,
         sem0, sem1, semi) = refs[nch:]
        core = lax.axis_index("c")
        sid = lax.axis_index("s")
        wid = core * 16 + sid
        row0 = pl.multiple_of(sid * ZPT, 8)
        ebase = pl.multiple_of(wid * ept, 8)

        def stage(j, cy):
            eoff = pl.multiple_of(ebase + j * EBLK, 8)
            pltpu.async_copy(dst_hbm.at[pl.ds(eoff, EBLK)], dstall.at[j], semi)
            if linear2:
                pltpu.async_copy(dst2_hbm.at[pl.ds(eoff, EBLK)],
                                 dst2all.at[j], semi)
            else:
                pltpu.async_copy(src_hbm.at[pl.ds(eoff, EBLK)],
                                 srcall.at[j], semi)
            return cy

        def drain(j, cy):
            eoff = pl.multiple_of(ebase + j * EBLK, 8)
            pltpu.make_async_copy(dst_hbm.at[pl.ds(eoff, EBLK)],
                                  dstall.at[j], semi).wait()
            if linear2:
                pltpu.make_async_copy(dst2_hbm.at[pl.ds(eoff, EBLK)],
                                      dst2all.at[j], semi).wait()
            else:
                pltpu.make_async_copy(src_hbm.at[pl.ds(eoff, EBLK)],
                                      srcall.at[j], semi).wait()
            return cy

        lax.fori_loop(0, nblk, stage, 0)
        lax.fori_loop(0, nblk, drain, 0)

        for c in range(npass):
            is_deg = want_deg and c == nch

            def start_gather(g, buf, sem):
                if linear2:
                    eoff = pl.multiple_of(ebase + g * EBLK, 8)
                    pltpu.async_copy(tbls[min(c, nch - 1)]
                                     .at[pl.ds(eoff, EBLK)], buf, sem)
                else:
                    pltpu.async_copy(tbls[min(c, nch - 1)].at[srcall.at[g]],
                                     buf, sem)

            def wait_gather(buf, sem):
                if linear2:
                    pltpu.make_async_copy(
                        tbls[min(c, nch - 1)].at[pl.ds(0, EBLK)], buf,
                        sem).wait()
                else:
                    pltpu.make_async_copy(
                        tbls[min(c, nch - 1)].at[srcall.at[0]], buf,
                        sem).wait()

            def scat(buf, g, idx2d):
                pltpu.sync_copy(buf, acc.at[idx2d.at[g]], add=True)

            pltpu.sync_copy(zeros_hbm, rows1)
            for r in range(ZPT // EBLK):
                pltpu.sync_copy(rows1, acc.at[pl.ds(row0 + r * EBLK, EBLK)])
            plsc.subcore_barrier()

            if is_deg:
                pltpu.sync_copy(ones_hbm, rows0)

                def dblk(g, cy):
                    scat(rows0, g, dstall)
                    return cy

                lax.fori_loop(0, nblk, dblk, 0)
            else:
                start_gather(0, rows0, sem0)

                def pblk(i, cy):
                    g0 = 2 * i
                    g1 = 2 * i + 1
                    start_gather(g1, rows1, sem1)
                    wait_gather(rows0, sem0)
                    scat(rows0, g0, dstall)
                    if linear2:
                        scat(rows0, g0, dst2all)

                    @pl.when(g0 + 2 < nblk)
                    def _():
                        start_gather(g0 + 2, rows0, sem0)

                    wait_gather(rows1, sem1)
                    scat(rows1, g1, dstall)
                    if linear2:
                        scat(rows1, g1, dst2all)
                    return cy

                lax.fori_loop(0, nblk // 2, pblk, 0)
            plsc.subcore_barrier()
            if is_deg:
                doff = pl.multiple_of(core * NR_ACC + row0, 8)
                pltpu.sync_copy(acc.at[pl.ds(row0, ZPT)],
                                deg_hbm.at[pl.ds(doff, ZPT)])
            else:
                poff = pl.multiple_of((core * nch + c) * NR_ACC + row0, 8)
                pltpu.sync_copy(acc.at[pl.ds(row0, ZPT)],
                                p_hbm.at[pl.ds(poff, ZPT)])

    out_type = [jax.ShapeDtypeStruct((2 * nch * NR_ACC, 128), jnp.float32),
                jax.ShapeDtypeStruct((2 * NR_ACC, 128), jnp.float32)]
    scratch = [
        pltpu.VMEM_SHARED((NR_ACC, 128), jnp.float32),
        pltpu.VMEM((nblk, EBLK), jnp.int32),
        pltpu.VMEM((nblk, EBLK), jnp.int32),
        pltpu.VMEM((nblk, EBLK), jnp.int32),
        pltpu.VMEM((EBLK, 128), jnp.float32),
        pltpu.VMEM((EBLK, 128), jnp.float32),
        pltpu.SemaphoreType.DMA,
        pltpu.SemaphoreType.DMA,
        pltpu.SemaphoreType.DMA,
    ]
    return mesh, body, out_type, scratch


@functools.partial(jax.jit, static_argnums=(4, 5))
def _sc_segsum_call(tbls, src_pad, dst_pad, dst2_pad, want_deg, linear2):
    nch = len(tbls)
    ept = dst_pad.shape[0] // NTILE
    mesh, body, out_type, scratch = _sc_gather_segsum(nch, want_deg, ept,
                                                      linear2)
    zeros = jnp.zeros((EBLK, 128), jnp.float32)
    ones = jnp.ones((EBLK, 128), jnp.float32)
    k = pl.kernel(body, out_type=out_type, mesh=mesh, scratch_types=scratch)
    p, deg = k(*tbls, src_pad, dst_pad, dst2_pad, zeros, ones)
    return (p.reshape(2, nch, NR_ACC, 128), deg.reshape(2, NR_ACC, 128))


def _segsum_rows(rows, dst, num_segments):
    return jnp.zeros((num_segments, rows.shape[1]), jnp.float32).at[dst].add(rows)


def kernel(x, et, H, raw_edge_index, lg_edge_index, W_tin, W_tself, W_tnb,
           W_etn, W_eg, W_s1_self, W_s1_nb, W_s2_self, W_s2_nb, a_n, a_e,
           W_mix, W_out):
    nblk_e = -(-E // (NTILE * EBLK))
    nblk_e += nblk_e % 2
    n_epad = NTILE * EBLK * nblk_e

    h = k1_h(et, W_tin)

    src = lg_edge_index[0]
    dst = lg_edge_index[1]
    agg = _segsum_rows(h[src], dst, E)
    deg = jnp.zeros((E,), jnp.float32).at[dst].add(1.0)
    deg16 = jnp.broadcast_to(deg[:, None], (E, 16))

    t_ch = k2_tsae(agg, deg16, h, W_tnb, W_tself, n_epad)

    h0_pad = _pad_idx(H[0], n_epad, N)
    h1_pad = _pad_idx(H[1], n_epad, N)
    P, _ = _sc_segsum_call(tuple(t_ch), h0_pad, h0_pad, h1_pad, False, True)

    edge_repr = k3_edge_repr(P, W_etn, W_eg)

    rs = raw_edge_index[0]
    rd = raw_edge_index[1]
    rs_pad = _pad_idx(rs, n_epad, 0)
    rd_pad = _pad_idx(rd, n_epad, N)
    x_ch = x.reshape(N, 2, 128).transpose(1, 0, 2)
    Q, rdeg_p = _sc_segsum_call((x_ch[0], x_ch[1]), rs_pad, rd_pad, rd_pad,
                                True, False)

    h1, *h1_ch = k4_h1(x, Q, rdeg_p, W_s1_self, W_s1_nb)

    Rm2, _ = _sc_segsum_call(tuple(h1_ch), rs_pad, rd_pad, rd_pad,
                             False, False)

    return k5_head(h1, Rm2, rdeg_p, edge_repr, W_s2_self, W_s2_nb,
                   a_n, a_e, W_mix, W_out)


# speedup vs baseline: 1.2449x; 1.1393x over previous
import functools

import jax
import jax.numpy as jnp
from jax import lax
from jax.experimental import pallas as pl
from jax.experimental.pallas import tpu as pltpu
from jax.experimental.pallas import tpu_sc as plsc

N = 10000
E = 160000
E_LG = 320000
F_IN = 256
H_DIM = 512
OUT = 256

RB_E = 1600
RB_N = 2000


def _leaky(x):
    return jnp.where(x >= 0, x, 0.2 * x)


def _k1_body(et_ref, w_ref, out_ref):
    out_ref[...] = _leaky(jnp.dot(et_ref[...], w_ref[...],
                                  preferred_element_type=jnp.float32)
                          ).astype(jnp.bfloat16)


def k1_h(et, W_tin):
    R = RB_E
    return pl.pallas_call(
        _k1_body,
        grid=(E // R,),
        in_specs=[
            pl.BlockSpec((R, 64), lambda i: (i, 0)),
            pl.BlockSpec((64, H_DIM), lambda i: (0, 0)),
        ],
        out_specs=pl.BlockSpec((R, H_DIM), lambda i: (i, 0)),
        out_shape=jax.ShapeDtypeStruct((E, H_DIM), jnp.bfloat16),
    )(et, W_tin)


def _k2_body(agg_ref, deg_ref, h_ref, wnb_ref, wself_ref, *out_refs):
    rrecip = 1.0 / jnp.maximum(deg_ref[:, 0:1], 1.0)
    aggn = agg_ref[...].astype(jnp.float32) * rrecip
    h = h_ref[...].astype(jnp.float32)
    for c in range(4):
        cs = slice(c * 128, (c + 1) * 128)
        out_refs[c][...] = jnp.maximum(
            jnp.dot(aggn, wnb_ref[:, cs], preferred_element_type=jnp.float32)
            + jnp.dot(h, wself_ref[:, cs], preferred_element_type=jnp.float32),
            0.0)


def k2_tsae(agg, deg16, h, W_tnb, W_tself, epad):
    R = RB_E
    return pl.pallas_call(
        _k2_body,
        grid=(E // R,),
        in_specs=[
            pl.BlockSpec((R, H_DIM), lambda i: (i, 0)),
            pl.BlockSpec((R, 16), lambda i: (i, 0)),
            pl.BlockSpec((R, H_DIM), lambda i: (i, 0)),
            pl.BlockSpec((H_DIM, H_DIM), lambda i: (0, 0)),
            pl.BlockSpec((H_DIM, H_DIM), lambda i: (0, 0)),
        ],
        out_specs=[pl.BlockSpec((R, 128), lambda i: (i, 0))] * 4,
        out_shape=[jax.ShapeDtypeStruct((epad, 128), jnp.float32)] * 4,
    )(agg, deg16, h, W_tnb, W_tself)


def _k3_body(p_ref, wetn_ref, weg_ref, out_ref):
    acc = None
    for c in range(4):
        nfe_c = p_ref[0, c] + p_ref[1, c]
        ks = slice(c * 128, (c + 1) * 128)
        term = jnp.dot(nfe_c, wetn_ref[ks, :], preferred_element_type=jnp.float32)
        acc = term if acc is None else acc + term
    t = _leaky(acc)
    out_ref[...] = jnp.dot(t, weg_ref[...], preferred_element_type=jnp.float32)


def k3_edge_repr(P, W_etn, W_eg):
    R = RB_N
    return pl.pallas_call(
        _k3_body,
        grid=(N // R,),
        in_specs=[
            pl.BlockSpec((2, 4, R, 128), lambda i: (0, 0, i, 0)),
            pl.BlockSpec((H_DIM, H_DIM), lambda i: (0, 0)),
            pl.BlockSpec((H_DIM, H_DIM), lambda i: (0, 0)),
        ],
        out_specs=pl.BlockSpec((R, H_DIM), lambda i: (i, 0)),
        out_shape=jax.ShapeDtypeStruct((N, H_DIM), jnp.float32),
    )(P, W_etn, W_eg)


def _k4_body(x_ref, q_ref, rdeg_ref, wself_ref, wnb_ref, out_ref, *outch_refs):
    rrecip = 1.0 / jnp.maximum(rdeg_ref[0, :, 0:1] + rdeg_ref[1, :, 0:1], 1.0)
    acc = jnp.dot(x_ref[...], wself_ref[...], preferred_element_type=jnp.float32)
    for c in range(2):
        m1_c = (q_ref[0, c] + q_ref[1, c]) * rrecip
        ks = slice(c * 128, (c + 1) * 128)
        acc = acc + jnp.dot(m1_c, wnb_ref[ks, :], preferred_element_type=jnp.float32)
    h1 = jnp.maximum(acc, 0.0)
    out_ref[...] = h1
    for c in range(4):
        outch_refs[c][...] = h1[:, c * 128:(c + 1) * 128]


def k4_h1(x, Q, rdeg, W_s1_self, W_s1_nb):
    R = RB_N
    return pl.pallas_call(
        _k4_body,
        grid=(N // R,),
        in_specs=[
            pl.BlockSpec((R, F_IN), lambda i: (i, 0)),
            pl.BlockSpec((2, 2, R, 128), lambda i: (0, 0, i, 0)),
            pl.BlockSpec((2, R, 128), lambda i: (0, i, 0)),
            pl.BlockSpec((F_IN, H_DIM), lambda i: (0, 0)),
            pl.BlockSpec((F_IN, H_DIM), lambda i: (0, 0)),
        ],
        out_specs=[pl.BlockSpec((R, H_DIM), lambda i: (i, 0))]
        + [pl.BlockSpec((R, 128), lambda i: (i, 0))] * 4,
        out_shape=[jax.ShapeDtypeStruct((N, H_DIM), jnp.float32)]
        + [jax.ShapeDtypeStruct((N, 128), jnp.float32)] * 4,
    )(x, Q, rdeg, W_s1_self, W_s1_nb)


def _k5_body(h1_ref, r_ref, rdeg_ref, er_ref, wself_ref, wnb_ref,
             an_ref, ae_ref, wmix_ref, wout_ref, out_ref):
    rrecip = 1.0 / jnp.maximum(rdeg_ref[0, :, 0:1] + rdeg_ref[1, :, 0:1], 1.0)
    acc = jnp.dot(h1_ref[...], wself_ref[...], preferred_element_type=jnp.float32)
    for c in range(4):
        m2_c = (r_ref[0, c] + r_ref[1, c]) * rrecip
        ks = slice(c * 128, (c + 1) * 128)
        acc = acc + jnp.dot(m2_c, wnb_ref[ks, :], preferred_element_type=jnp.float32)
    node = jnp.maximum(acc, 0.0)
    edge = er_ref[...]
    sn = jnp.dot(node, an_ref[...], preferred_element_type=jnp.float32)
    se = jnp.dot(edge, ae_ref[...], preferred_element_type=jnp.float32)
    m = jnp.maximum(sn, se)
    en = jnp.exp(sn - m)
    ee = jnp.exp(se - m)
    inv = 1.0 / (en + ee)
    mixed = (en * inv) * node + (ee * inv) * edge
    o = jnp.dot(mixed, wmix_ref[...], preferred_element_type=jnp.float32)
    logits = jnp.dot(o, wout_ref[...], preferred_element_type=jnp.float32)
    mx = jnp.max(logits, axis=1, keepdims=True)
    lse = mx + jnp.log(jnp.sum(jnp.exp(logits - mx), axis=1, keepdims=True))
    out_ref[...] = logits - lse


def k5_head(h1, Rm2, rdeg, edge_repr, W_s2_self, W_s2_nb, a_n, a_e, W_mix, W_out):
    a_n = a_n.reshape(H_DIM, 1)
    a_e = a_e.reshape(H_DIM, 1)
    R = RB_N
    return pl.pallas_call(
        _k5_body,
        grid=(N // R,),
        in_specs=[
            pl.BlockSpec((R, H_DIM), lambda i: (i, 0)),
            pl.BlockSpec((2, 4, R, 128), lambda i: (0, 0, i, 0)),
            pl.BlockSpec((2, R, 128), lambda i: (0, i, 0)),
            pl.BlockSpec((R, H_DIM), lambda i: (i, 0)),
            pl.BlockSpec((H_DIM, H_DIM), lambda i: (0, 0)),
            pl.BlockSpec((H_DIM, H_DIM), lambda i: (0, 0)),
            pl.BlockSpec((H_DIM, 1), lambda i: (0, 0)),
            pl.BlockSpec((H_DIM, 1), lambda i: (0, 0)),
            pl.BlockSpec((H_DIM, H_DIM), lambda i: (0, 0)),
            pl.BlockSpec((H_DIM, OUT), lambda i: (0, 0)),
        ],
        out_specs=pl.BlockSpec((R, OUT), lambda i: (i, 0)),
        out_shape=jax.ShapeDtypeStruct((N, OUT), jnp.float32),
    )(h1, Rm2, rdeg, edge_repr, W_s2_self, W_s2_nb, a_n, a_e, W_mix, W_out)


NTILE = 32
EBLK = 64
NR_ACC = 10240
ZPT = NR_ACC // 16


def _pad_idx(idx, n_pad, pad_val):
    idx = idx.astype(jnp.int32)
    pad = jnp.full((n_pad - idx.shape[0],), pad_val, jnp.int32)
    return jnp.concatenate([idx, pad])


def _sc_gather_segsum(nch, want_deg, ept, linear2):
    mesh = plsc.VectorSubcoreMesh(core_axis_name="c", subcore_axis_name="s")
    nblk = ept // EBLK
    assert nblk % 2 == 0
    npass = nch + (1 if want_deg else 0)

    def body(*refs):
        tbls = refs[:nch]
        (src_hbm, dst_hbm, dst2_hbm, zeros_hbm, ones_hbm, p_hbm, deg_hbm,
         acc, srcall, dstall, dst2all, rows0, rows1,
         sem0, sem1, semi) = refs[nch:]
        core = lax.axis_index("c")
        sid = lax.axis_index("s")
        wid = core * 16 + sid
        row0 = pl.multiple_of(sid * ZPT, 8)
        ebase = pl.multiple_of(wid * ept, 8)

        def stage(j, cy):
            eoff = pl.multiple_of(ebase + j * EBLK, 8)
            pltpu.async_copy(dst_hbm.at[pl.ds(eoff, EBLK)], dstall.at[j], semi)
            if linear2:
                pltpu.async_copy(dst2_hbm.at[pl.ds(eoff, EBLK)],
                                 dst2all.at[j], semi)
            else:
                pltpu.async_copy(src_hbm.at[pl.ds(eoff, EBLK)],
                                 srcall.at[j], semi)
            return cy

        def drain(j, cy):
            eoff = pl.multiple_of(ebase + j * EBLK, 8)
            pltpu.make_async_copy(dst_hbm.at[pl.ds(eoff, EBLK)],
                                  dstall.at[j], semi).wait()
            if linear2:
                pltpu.make_async_copy(dst2_hbm.at[pl.ds(eoff, EBLK)],
                                      dst2all.at[j], semi).wait()
            else:
                pltpu.make_async_copy(src_hbm.at[pl.ds(eoff, EBLK)],
                                      srcall.at[j], semi).wait()
            return cy

        lax.fori_loop(0, nblk, stage, 0)
        lax.fori_loop(0, nblk, drain, 0)

        for c in range(npass):
            is_deg = want_deg and c == nch

            def start_gather(g, buf, sem):
                if linear2:
                    eoff = pl.multiple_of(ebase + g * EBLK, 8)
                    pltpu.async_copy(tbls[min(c, nch - 1)]
                                     .at[pl.ds(eoff, EBLK)], buf, sem)
                else:
                    pltpu.async_copy(tbls[min(c, nch - 1)].at[srcall.at[g]],
                                     buf, sem)

            def wait_gather(buf, sem):
                if linear2:
                    pltpu.make_async_copy(
                        tbls[min(c, nch - 1)].at[pl.ds(0, EBLK)], buf,
                        sem).wait()
                else:
                    pltpu.make_async_copy(
                        tbls[min(c, nch - 1)].at[srcall.at[0]], buf,
                        sem).wait()

            def scat(buf, g, idx2d):
                pltpu.sync_copy(buf, acc.at[idx2d.at[g]], add=True)

            pltpu.sync_copy(zeros_hbm, rows1)
            for r in range(ZPT // EBLK):
                pltpu.sync_copy(rows1, acc.at[pl.ds(row0 + r * EBLK, EBLK)])
            plsc.subcore_barrier()

            if is_deg:
                pltpu.sync_copy(ones_hbm, rows0)

                def dblk(g, cy):
                    scat(rows0, g, dstall)
                    return cy

                lax.fori_loop(0, nblk, dblk, 0)
            else:
                start_gather(0, rows0, sem0)

                def pblk(i, cy):
                    g0 = 2 * i
                    g1 = 2 * i + 1
                    start_gather(g1, rows1, sem1)
                    wait_gather(rows0, sem0)
                    scat(rows0, g0, dstall)
                    if linear2:
                        scat(rows0, g0, dst2all)

                    @pl.when(g0 + 2 < nblk)
                    def _():
                        start_gather(g0 + 2, rows0, sem0)

                    wait_gather(rows1, sem1)
                    scat(rows1, g1, dstall)
                    if linear2:
                        scat(rows1, g1, dst2all)
                    return cy

                lax.fori_loop(0, nblk // 2, pblk, 0)
            plsc.subcore_barrier()
            if is_deg:
                doff = pl.multiple_of(core * NR_ACC + row0, 8)
                pltpu.sync_copy(acc.at[pl.ds(row0, ZPT)],
                                deg_hbm.at[pl.ds(doff, ZPT)])
            else:
                poff = pl.multiple_of((core * nch + c) * NR_ACC + row0, 8)
                pltpu.sync_copy(acc.at[pl.ds(row0, ZPT)],
                                p_hbm.at[pl.ds(poff, ZPT)])

    out_type = [jax.ShapeDtypeStruct((2 * nch * NR_ACC, 128), jnp.float32),
                jax.ShapeDtypeStruct((2 * NR_ACC, 128), jnp.float32)]
    scratch = [
        pltpu.VMEM_SHARED((NR_ACC, 128), jnp.float32),
        pltpu.VMEM((nblk, EBLK), jnp.int32),
        pltpu.VMEM((nblk, EBLK), jnp.int32),
        pltpu.VMEM((nblk, EBLK), jnp.int32),
        pltpu.VMEM((EBLK, 128), jnp.float32),
        pltpu.VMEM((EBLK, 128), jnp.float32),
        pltpu.SemaphoreType.DMA,
        pltpu.SemaphoreType.DMA,
        pltpu.SemaphoreType.DMA,
    ]
    return mesh, body, out_type, scratch


@functools.partial(jax.jit, static_argnums=(4, 5))
def _sc_segsum_call(tbls, src_pad, dst_pad, dst2_pad, want_deg, linear2):
    nch = len(tbls)
    ept = dst_pad.shape[0] // NTILE
    mesh, body, out_type, scratch = _sc_gather_segsum(nch, want_deg, ept,
                                                      linear2)
    zeros = jnp.zeros((EBLK, 128), jnp.float32)
    ones = jnp.ones((EBLK, 128), jnp.float32)
    k = pl.kernel(body, out_type=out_type, mesh=mesh, scratch_types=scratch)
    p, deg = k(*tbls, src_pad, dst_pad, dst2_pad, zeros, ones)
    return (p.reshape(2, nch, NR_ACC, 128), deg.reshape(2, NR_ACC, 128))


def _segsum_rows(rows, dst, num_segments):
    return jnp.zeros((num_segments, rows.shape[1]), jnp.float32).at[dst].add(rows)


def kernel(x, et, H, raw_edge_index, lg_edge_index, W_tin, W_tself, W_tnb,
           W_etn, W_eg, W_s1_self, W_s1_nb, W_s2_self, W_s2_nb, a_n, a_e,
           W_mix, W_out):
    nblk_e = -(-E // (NTILE * EBLK))
    nblk_e += nblk_e % 2
    n_epad = NTILE * EBLK * nblk_e

    h = k1_h(et, W_tin)

    src = lg_edge_index[0]
    dst = lg_edge_index[1]
    agg = jnp.zeros((E, H_DIM), jnp.bfloat16).at[dst].add(h[src])
    deg = jnp.zeros((E,), jnp.float32).at[dst].add(1.0)
    deg16 = jnp.broadcast_to(deg[:, None], (E, 16))

    t_ch = k2_tsae(agg, deg16, h, W_tnb, W_tself, n_epad)

    h0_pad = _pad_idx(H[0], n_epad, N)
    h1_pad = _pad_idx(H[1], n_epad, N)
    P, _ = _sc_segsum_call(tuple(t_ch), h0_pad, h0_pad, h1_pad, False, True)

    edge_repr = k3_edge_repr(P, W_etn, W_eg)

    rs = raw_edge_index[0]
    rd = raw_edge_index[1]
    rs_pad = _pad_idx(rs, n_epad, 0)
    rd_pad = _pad_idx(rd, n_epad, N)
    x_ch = x.reshape(N, 2, 128).transpose(1, 0, 2)
    Q, rdeg_p = _sc_segsum_call((x_ch[0], x_ch[1]), rs_pad, rd_pad, rd_pad,
                                True, False)

    h1, *h1_ch = k4_h1(x, Q, rdeg_p, W_s1_self, W_s1_nb)

    Rm2, _ = _sc_segsum_call(tuple(h1_ch), rs_pad, rd_pad, rd_pad,
                             False, False)

    return k5_head(h1, Rm2, rdeg_p, edge_repr, W_s2_self, W_s2_nb,
                   a_n, a_e, W_mix, W_out)
